# Initial kernel scaffold; baseline (speedup 1.0000x reference)
import functools

import jax
import jax.numpy as jnp
from jax import lax
from jax.experimental import pallas as pl
from jax.experimental.pallas import tpu as pltpu
from jax.experimental.pallas import tpu_sc as plsc

N = 320000
C = 10000
H = 64
D_IN = 128

NW = 32
P = N // NW
TILE = 400
NT = P // TILE
KCAP = 64
GCH = 80
NG = P // GCH
NEG = jnp.float32(-3.0e38)

_MESH = plsc.VectorSubcoreMesh(core_axis_name="c", subcore_axis_name="s")


def _ln_relu_mm(pre, g, be, W2, b2):
    mu = jnp.mean(pre, axis=-1, keepdims=True)
    var = jnp.mean((pre - mu) ** 2, axis=-1, keepdims=True)
    h = (pre - mu) * lax.rsqrt(var + 1e-5) * g + be
    h = jnp.maximum(h, 0.0)
    return jnp.dot(h, W2, preferred_element_type=jnp.float32) + b2


def _mlp0_kern(x_ref, W1, b1, g, be, W2, b2, out_ref):
    pre = jnp.dot(x_ref[...], W1[...], preferred_element_type=jnp.float32) + b1[...]
    out_ref[...] = _ln_relu_mm(pre, g[...], be[...], W2[...], b2[...])


def _mlpcat_kern(h_ref, gv_ref, W1a, W1b, b1, g, be, W2, b2, out_ref):
    pre = (jnp.dot(h_ref[...], W1a[...], preferred_element_type=jnp.float32)
           + jnp.dot(gv_ref[...], W1b[...], preferred_element_type=jnp.float32)
           + b1[...])
    out_ref[...] = _ln_relu_mm(pre, g[...], be[...], W2[...], b2[...])


def _catlin_kern(h_ref, gv_ref, Wa, Wb, b, out_ref):
    out_ref[...] = (jnp.dot(h_ref[...], Wa[...], preferred_element_type=jnp.float32)
                    + jnp.dot(gv_ref[...], Wb[...], preferred_element_type=jnp.float32)
                    + b[...])


RB = 512
GRID = N // RB

_w = lambda r, c: pl.BlockSpec((r, c), lambda i: (0, 0))
_rows = lambda c: pl.BlockSpec((RB, c), lambda i: (i, 0))

_mlp0 = pl.pallas_call(
    _mlp0_kern,
    grid=(GRID,),
    in_specs=[_rows(D_IN), _w(D_IN, H), _w(1, H), _w(1, H), _w(1, H),
              _w(H, H), _w(1, H)],
    out_specs=_rows(H),
    out_shape=jax.ShapeDtypeStruct((N, H), jnp.float32),
)

_mlpcat = pl.pallas_call(
    _mlpcat_kern,
    grid=(GRID,),
    in_specs=[_rows(H), _rows(H), _w(H, H), _w(H, H), _w(1, H), _w(1, H),
              _w(1, H), _w(H, H), _w(1, H)],
    out_specs=_rows(H),
    out_shape=jax.ShapeDtypeStruct((N, H), jnp.float32),
)

_catlin = pl.pallas_call(
    _catlin_kern,
    grid=(GRID,),
    in_specs=[_rows(H), _rows(H), _w(H, H), _w(H, H), _w(1, H)],
    out_specs=_rows(H),
    out_shape=jax.ShapeDtypeStruct((N, H), jnp.float32),
)


def _combine_bnd(bndr, bndi):
    rows = bndr
    ids = bndi
    for sh in (1, 2, 4, 8, 16, 32):
        prev_id = jnp.concatenate(
            [jnp.full((sh, ids.shape[1]), -9, jnp.int32), ids[:-sh]], axis=0)
        prev_r = jnp.concatenate(
            [jnp.full((sh, H), NEG, jnp.float32), rows[:-sh]], axis=0)
        same = (ids == prev_id)[:, :1]
        rows = jnp.where(same, jnp.maximum(rows, prev_r), rows)
    return rows


def _apply_patches(out_ref, rows, bndi_smem):
    for j in range(2 * NW):
        idj = bndi_smem[j, 0]
        nxt = bndi_smem[j + 1, 0] if j < 2 * NW - 1 else jnp.int32(-7)

        @pl.when((idj >= 0) & (idj != nxt))
        def _():
            out_ref[pl.ds(idj, 1), :] = rows[j:j + 1, :]


def _patch_kern(m_ref, bndr_ref, bndi_ref, bndi_smem, out_ref):
    out_ref[...] = m_ref[0:C, :]
    rows = _combine_bnd(bndr_ref[...], bndi_ref[...])
    _apply_patches(out_ref, rows, bndi_smem)


def _patch_norm_kern(m_ref, bndr_ref, bndi_ref, bndi_smem, pres_ref, out_ref):
    out_ref[...] = m_ref[0:C, :]
    rows = _combine_bnd(bndr_ref[...], bndi_ref[...])
    _apply_patches(out_ref, rows, bndi_smem)
    h = out_ref[...]
    nrm = jnp.sqrt(jnp.sum(h * h, axis=1, keepdims=True))
    ok = jnp.sum(pres_ref[...], axis=1, keepdims=True) > 0
    out_ref[...] = jnp.where(ok, h / jnp.maximum(nrm, 1e-12), 0.0)


def _full(shp):
    n = len(shp)
    return pl.BlockSpec(shp, lambda: (0,) * n)


_patch = pl.pallas_call(
    _patch_kern,
    in_specs=[_full((C + 1, H)), _full((2 * NW, H)), _full((2 * NW, 16)),
              pl.BlockSpec(memory_space=pltpu.SMEM)],
    out_specs=_full((C, H)),
    out_shape=jax.ShapeDtypeStruct((C, H), jnp.float32),
)

_patch_norm = pl.pallas_call(
    _patch_norm_kern,
    in_specs=[_full((C + 1, H)), _full((2 * NW, H)), _full((2 * NW, 16)),
              pl.BlockSpec(memory_space=pltpu.SMEM), _full((C, NW))],
    out_specs=_full((C, H)),
    out_shape=jax.ShapeDtypeStruct((C, H), jnp.float32),
)


def _splat_i32(v):
    return jnp.full((16,), v, jnp.int32)


@functools.partial(
    pl.kernel,
    out_type=(
        jax.ShapeDtypeStruct((C + 1, H), jnp.float32),
        jax.ShapeDtypeStruct((2 * NW * H,), jnp.float32),
        jax.ShapeDtypeStruct((2 * NW * 16,), jnp.int32),
        jax.ShapeDtypeStruct((NW * C,), jnp.int32),
    ),
    mesh=_MESH,
    scratch_types=[
        pltpu.VMEM((TILE * H,), jnp.float32),
        pltpu.VMEM((TILE,), jnp.int32),
        pltpu.VMEM((KCAP, H), jnp.float32),
        pltpu.VMEM((KCAP,), jnp.int32),
        pltpu.VMEM((C,), jnp.int32),
        pltpu.VMEM((2 * H,), jnp.float32),
        pltpu.VMEM((32,), jnp.int32),
        pltpu.VMEM((16,), jnp.int32),
        pltpu.SemaphoreType.DMA,
    ],
)
def _sc_segmax(h_hbm, cl_hbm, m_hbm, bndr_hbm, bndi_hbm, pres_hbm,
               htile, idst, stg, stgi, presl, bndrl, bndil, idbuf, sem):
    wid = lax.axis_index("s") * 2 + lax.axis_index("c")
    row0 = wid * P
    iota = lax.iota(jnp.int32, 16)

    def _reset_stgi():
        for j in range(KCAP // 16):
            stgi[pl.ds(j * 16, 16)] = _splat_i32(C)

    _reset_stgi()
    for j in range(2):
        bndil[pl.ds(j * 16, 16)] = _splat_i32(-1)
    for j in range(2 * H // 16):
        bndrl[pl.ds(j * 16, 16)] = jnp.full((16,), NEG, jnp.float32)

    def _zp(i, c):
        presl[pl.ds(i * 16, 16)] = jnp.zeros((16,), jnp.int32)
        return c
    lax.fori_loop(0, C // 16, _zp, 0)

    def _left():
        pltpu.sync_copy(cl_hbm.at[pl.ds(row0 - 16, 16)], idbuf)
        return idbuf[15]

    left_id = lax.cond(wid > 0, _left, lambda: jnp.int32(-1))

    def _right():
        pltpu.sync_copy(cl_hbm.at[pl.ds(row0 + P, 16)], idbuf)
        return idbuf[0]

    right_id = lax.cond(wid < NW - 1, _right, lambda: jnp.int32(-1))

    def _flush():
        pltpu.async_copy(stg, m_hbm.at[stgi], sem).wait()
        _reset_stgi()

    def _append(kp, cur_id, a0, a1, a2, a3):
        rk = _splat_i32(kp)
        plsc.store_scatter(stg, (rk, iota), a0)
        plsc.store_scatter(stg, (rk, iota + 16), a1)
        plsc.store_scatter(stg, (rk, iota + 32), a2)
        plsc.store_scatter(stg, (rk, iota + 48), a3)
        plsc.store_scatter(stgi, (rk,), _splat_i32(cur_id))
        return kp + 1

    def _to_bnd(slot, cur_id, a0, a1, a2, a3):
        bndil[pl.ds(slot * 16, 16)] = _splat_i32(cur_id)
        bndrl[pl.ds(slot * H, 16)] = a0
        bndrl[pl.ds(slot * H + 16, 16)] = a1
        bndrl[pl.ds(slot * H + 32, 16)] = a2
        bndrl[pl.ds(slot * H + 48, 16)] = a3

    def tile_body(t, carry):
        base = row0 + t * TILE
        pltpu.sync_copy(h_hbm.at[pl.ds(base * H, TILE * H)], htile)
        pltpu.sync_copy(cl_hbm.at[pl.ds(base, TILE)], idst)

        ones = jnp.ones((16,), jnp.int32)

        def pres_body(gi, c):
            idv = idst[pl.ds(gi * 16, 16)]
            plsc.store_scatter(presl, (idv,), ones)
            return c
        lax.fori_loop(0, TILE // 16, pres_body, 0)

        def row_body(r, c2):
            cur_id, cnt, kpos, a0, a1, a2, a3 = c2
            rid = idst[r]
            off = r * H
            v0 = htile[pl.ds(off, 16)]
            v1 = htile[pl.ds(off + 16, 16)]
            v2 = htile[pl.ds(off + 32, 16)]
            v3 = htile[pl.ds(off + 48, 16)]
            new = rid != cur_id

            def do_close(cnt, kpos):
                shared_first = (cnt == 0) & (cur_id == left_id)

                def tb(kp):
                    _to_bnd(0, cur_id, a0, a1, a2, a3)
                    return kp

                def tm(kp):
                    return _append(kp, cur_id, a0, a1, a2, a3)

                kpos2 = lax.cond(shared_first, tb, tm, kpos)
                return cnt + 1, kpos2

            cnt, kpos = lax.cond(new & (cur_id >= 0), do_close,
                                 lambda c, k: (c, k), cnt, kpos)

            def fl(k):
                _flush()
                return jnp.int32(0)

            kpos = lax.cond(kpos == KCAP, fl, lambda k: k, kpos)

            a0 = jnp.where(new, v0, jnp.maximum(a0, v0))
            a1 = jnp.where(new, v1, jnp.maximum(a1, v1))
            a2 = jnp.where(new, v2, jnp.maximum(a2, v2))
            a3 = jnp.where(new, v3, jnp.maximum(a3, v3))
            return (rid, cnt, kpos, a0, a1, a2, a3)

        return lax.fori_loop(0, TILE, row_body, carry)

    neg16 = jnp.full((16,), NEG, jnp.float32)
    init = (jnp.int32(-2), jnp.int32(0), jnp.int32(0),
            neg16, neg16, neg16, neg16)
    cur_id, cnt, kpos, a0, a1, a2, a3 = lax.fori_loop(0, NT, tile_body, init)

    last_shared = (cur_id == right_id) | (cur_id == left_id)

    def end_bnd(kp):
        _to_bnd(1, cur_id, a0, a1, a2, a3)
        return kp

    def end_m(kp):
        return _append(kp, cur_id, a0, a1, a2, a3)

    kpos = lax.cond(last_shared, end_bnd, end_m, kpos)

    @pl.when(kpos > 0)
    def _():
        _flush()

    pltpu.sync_copy(bndrl, bndr_hbm.at[pl.ds(wid * 2 * H, 2 * H)])
    pltpu.sync_copy(bndil, bndi_hbm.at[pl.ds(wid * 32, 32)])
    pltpu.sync_copy(presl, pres_hbm.at[pl.ds(wid * C, C)])


@functools.partial(
    pl.kernel,
    out_type=jax.ShapeDtypeStruct((N, H), jnp.float32),
    mesh=_MESH,
    scratch_types=[
        pltpu.VMEM((P,), jnp.int32),
        pltpu.VMEM((4, GCH, H), jnp.float32),
        pltpu.SemaphoreType.DMA,
        pltpu.SemaphoreType.DMA,
        pltpu.SemaphoreType.DMA,
        pltpu.SemaphoreType.DMA,
        pltpu.SemaphoreType.DMA,
        pltpu.SemaphoreType.DMA,
        pltpu.SemaphoreType.DMA,
        pltpu.SemaphoreType.DMA,
    ],
)
def _sc_gather(m_hbm, cl_hbm, out_hbm,
               idsl, ring, g0, g1, g2, g3, o0, o1, o2, o3):
    wid = lax.axis_index("s") * 2 + lax.axis_index("c")
    base = wid * P
    gsem = (g0, g1, g2, g3)
    osem = (o0, o1, o2, o3)

    pltpu.sync_copy(cl_hbm.at[pl.ds(base, P)], idsl)

    def round_body(r, c):
        cps = []
        for s in range(4):
            i = r * 4 + s

            @pl.when(r > 0)
            def _():
                pltpu.make_async_copy(
                    ring.at[s], out_hbm.at[pl.ds(base, GCH)], osem[s]).wait()

            idx = idsl.at[pl.ds(i * GCH, GCH)]
            cps.append(pltpu.async_copy(m_hbm.at[idx], ring.at[s], gsem[s]))
        for s in range(4):
            i = r * 4 + s
            cps[s].wait()
            pltpu.async_copy(
                ring.at[s], out_hbm.at[pl.ds(base + i * GCH, GCH)], osem[s])
        return c

    lax.fori_loop(0, NG // 4, round_body, 0)

    for s in range(4):
        pltpu.make_async_copy(
            ring.at[s], out_hbm.at[pl.ds(base, GCH)], osem[s]).wait()

    i = NG - 1
    idx = idsl.at[pl.ds(i * GCH, GCH)]
    pltpu.async_copy(m_hbm.at[idx], ring.at[0], g0).wait()
    pltpu.sync_copy(ring.at[0], out_hbm.at[pl.ds(base + i * GCH, GCH)])


def _seg_round(h, cl):
    mb, br, bi, pres = _sc_segmax(h.reshape(N * H), cl)
    bi2 = bi.reshape(2 * NW, 16)
    m = _patch(mb, br.reshape(2 * NW, H), bi2, bi2)
    return _sc_gather(m, cl)


def kernel(x, cluster, W1_0, b1_0, g_0, be_0, W2_0, b2_0,
           W1_1, b1_1, g_1, be_1, W2_1, b2_1,
           W1_2, b1_2, g_2, be_2, W2_2, b2_2, Wf, bf):
    r1 = lambda v: v.reshape(1, H)
    cl = cluster

    h1 = _mlp0(x, W1_0, r1(b1_0), r1(g_0), r1(be_0), W2_0, r1(b2_0))
    gv1 = _seg_round(h1, cl)

    h2 = _mlpcat(h1, gv1, W1_1[:H], W1_1[H:], r1(b1_1), r1(g_1), r1(be_1),
                 W2_1, r1(b2_1))
    gv2 = _seg_round(h2, cl)

    h3 = _mlpcat(h2, gv2, W1_2[:H], W1_2[H:], r1(b1_2), r1(g_2), r1(be_2),
                 W2_2, r1(b2_2))
    gv3 = _seg_round(h3, cl)

    f = _catlin(h3, gv3, Wf[:H], Wf[H:], r1(bf))
    mb, br, bi, pres = _sc_segmax(f.reshape(N * H), cl)
    bi2 = bi.reshape(2 * NW, 16)
    out = _patch_norm(mb, br.reshape(2 * NW, H), bi2, bi2,
                      pres.reshape(NW, C).T)
    return out


# speedup vs baseline: 1.1658x; 1.1658x over previous
import functools

import jax
import jax.numpy as jnp
from jax import lax
from jax.experimental import pallas as pl
from jax.experimental.pallas import tpu as pltpu
from jax.experimental.pallas import tpu_sc as plsc

N = 320000
C = 10000
H = 64
D_IN = 128

NW = 32
P = N // NW
TILE = 400
NT = P // TILE
KCAP = 64
GCH = 80
NG = P // GCH
NEG = -3.0e38

_MESH = plsc.VectorSubcoreMesh(core_axis_name="c", subcore_axis_name="s")


def _ln_relu_mm(pre, g, be, W2, b2):
    mu = jnp.mean(pre, axis=-1, keepdims=True)
    var = jnp.mean((pre - mu) ** 2, axis=-1, keepdims=True)
    h = (pre - mu) * lax.rsqrt(var + 1e-5) * g + be
    h = jnp.maximum(h, 0.0)
    return jnp.dot(h, W2, preferred_element_type=jnp.float32) + b2


def _mlp0_kern(x_ref, W1, b1, g, be, W2, b2, out_ref):
    pre = jnp.dot(x_ref[...], W1[...], preferred_element_type=jnp.float32) + b1[...]
    out_ref[...] = _ln_relu_mm(pre, g[...], be[...], W2[...], b2[...])


def _mlpcat_kern(h_ref, gv_ref, W1a, W1b, b1, g, be, W2, b2, out_ref):
    pre = (jnp.dot(h_ref[...], W1a[...], preferred_element_type=jnp.float32)
           + jnp.dot(gv_ref[...], W1b[...], preferred_element_type=jnp.float32)
           + b1[...])
    out_ref[...] = _ln_relu_mm(pre, g[...], be[...], W2[...], b2[...])


def _catlin_kern(h_ref, gv_ref, Wa, Wb, b, out_ref):
    out_ref[...] = (jnp.dot(h_ref[...], Wa[...], preferred_element_type=jnp.float32)
                    + jnp.dot(gv_ref[...], Wb[...], preferred_element_type=jnp.float32)
                    + b[...])


RB = 512
GRID = N // RB

_w = lambda r, c: pl.BlockSpec((r, c), lambda i: (0, 0))
_rows = lambda c: pl.BlockSpec((RB, c), lambda i: (i, 0))

_mlp0 = pl.pallas_call(
    _mlp0_kern,
    grid=(GRID,),
    in_specs=[_rows(D_IN), _w(D_IN, H), _w(1, H), _w(1, H), _w(1, H),
              _w(H, H), _w(1, H)],
    out_specs=_rows(H),
    out_shape=jax.ShapeDtypeStruct((N, H), jnp.float32),
)

_mlpcat = pl.pallas_call(
    _mlpcat_kern,
    grid=(GRID,),
    in_specs=[_rows(H), _rows(H), _w(H, H), _w(H, H), _w(1, H), _w(1, H),
              _w(1, H), _w(H, H), _w(1, H)],
    out_specs=_rows(H),
    out_shape=jax.ShapeDtypeStruct((N, H), jnp.float32),
)

_catlin = pl.pallas_call(
    _catlin_kern,
    grid=(GRID,),
    in_specs=[_rows(H), _rows(H), _w(H, H), _w(H, H), _w(1, H)],
    out_specs=_rows(H),
    out_shape=jax.ShapeDtypeStruct((N, H), jnp.float32),
)


def _combine_bnd(bndr, bndi):
    rows = bndr
    ids = bndi
    for sh in (1, 2, 4, 8, 16, 32):
        prev_id = jnp.concatenate(
            [jnp.full((sh, ids.shape[1]), -9, jnp.int32), ids[:-sh]], axis=0)
        prev_r = jnp.concatenate(
            [jnp.full((sh, H), NEG, jnp.float32), rows[:-sh]], axis=0)
        same = (ids == prev_id)[:, :1]
        rows = jnp.where(same, jnp.maximum(rows, prev_r), rows)
    return rows


def _apply_patches(out_ref, rows, bndi_smem):
    for j in range(2 * NW):
        idj = bndi_smem[j, 0]
        nxt = bndi_smem[j + 1, 0] if j < 2 * NW - 1 else jnp.int32(-7)

        @pl.when((idj >= 0) & (idj != nxt))
        def _():
            out_ref[pl.ds(idj, 1), :] = rows[j:j + 1, :]


def _patch_kern(m_ref, bndr_ref, bndi_ref, bndi_smem, out_ref):
    out_ref[...] = m_ref[0:C, :]
    rows = _combine_bnd(bndr_ref[...], bndi_ref[...])
    _apply_patches(out_ref, rows, bndi_smem)


def _patch_norm_kern(m_ref, bndr_ref, bndi_ref, bndi_smem, pres_ref, out_ref):
    out_ref[...] = m_ref[0:C, :]
    rows = _combine_bnd(bndr_ref[...], bndi_ref[...])
    _apply_patches(out_ref, rows, bndi_smem)
    h = out_ref[...]
    nrm = jnp.sqrt(jnp.sum(h * h, axis=1, keepdims=True))
    ok = jnp.sum(pres_ref[...], axis=1, keepdims=True) > 0
    out_ref[...] = jnp.where(ok, h / jnp.maximum(nrm, 1e-12), 0.0)


def _full(shp):
    n = len(shp)
    return pl.BlockSpec(shp, lambda: (0,) * n)


_patch = pl.pallas_call(
    _patch_kern,
    in_specs=[_full((C + 1, H)), _full((2 * NW, H)), _full((2 * NW, 16)),
              pl.BlockSpec(memory_space=pltpu.SMEM)],
    out_specs=_full((C, H)),
    out_shape=jax.ShapeDtypeStruct((C, H), jnp.float32),
)

_patch_norm = pl.pallas_call(
    _patch_norm_kern,
    in_specs=[_full((C + 1, H)), _full((2 * NW, H)), _full((2 * NW, 16)),
              pl.BlockSpec(memory_space=pltpu.SMEM), _full((C, NW))],
    out_specs=_full((C, H)),
    out_shape=jax.ShapeDtypeStruct((C, H), jnp.float32),
)


def _splat_i32(v):
    return jnp.full((16,), v, jnp.int32)


@functools.partial(
    pl.kernel,
    out_type=(
        jax.ShapeDtypeStruct((C + 1, H), jnp.float32),
        jax.ShapeDtypeStruct((2 * NW * H,), jnp.float32),
        jax.ShapeDtypeStruct((2 * NW * 16,), jnp.int32),
        jax.ShapeDtypeStruct((NW * C,), jnp.int32),
    ),
    mesh=_MESH,
    compiler_params=pltpu.CompilerParams(needs_layout_passes=False, use_tc_tiling_on_sc=False),
    scratch_types=[
        pltpu.VMEM((TILE * H,), jnp.float32),
        pltpu.VMEM((TILE,), jnp.int32),
        pltpu.VMEM((KCAP, H), jnp.float32),
        pltpu.VMEM((KCAP,), jnp.int32),
        pltpu.VMEM((10112,), jnp.int32),
        pltpu.VMEM((2 * H,), jnp.float32),
        pltpu.VMEM((32,), jnp.int32),
        pltpu.VMEM((16,), jnp.int32),
        pltpu.SemaphoreType.DMA,
    ],
)
def _sc_segmax(h_hbm, cl_hbm, m_hbm, bndr_hbm, bndi_hbm, pres_hbm,
               htile, idst, stg, stgi, presl, bndrl, bndil, idbuf, sem):
    wid = lax.axis_index("s") * 2 + lax.axis_index("c")
    row0 = wid * P
    iota = lax.iota(jnp.int32, 16)

    def _reset_stgi():
        for j in range(KCAP // 16):
            stgi[pl.ds(j * 16, 16)] = _splat_i32(C)

    _reset_stgi()
    for j in range(2):
        bndil[pl.ds(j * 16, 16)] = _splat_i32(-1)
    for j in range(2 * H // 16):
        bndrl[pl.ds(j * 16, 16)] = jnp.full((16,), NEG, jnp.float32)

    def _zp(i, c):
        presl[pl.ds(i * 16, 16)] = jnp.zeros((16,), jnp.int32)
        return c
    lax.fori_loop(0, C // 16, _zp, 0)

    def _left():
        pltpu.sync_copy(cl_hbm.at[pl.ds(row0 - 16, 16)], idbuf)
        return idbuf[pl.ds(0, 16)][15]

    left_id = lax.cond(wid > 0, _left, lambda: jnp.int32(-1))

    def _right():
        pltpu.sync_copy(cl_hbm.at[pl.ds(row0 + P, 16)], idbuf)
        return idbuf[pl.ds(0, 16)][0]

    right_id = lax.cond(wid < NW - 1, _right, lambda: jnp.int32(-1))

    def _flush():
        pltpu.async_copy(stg, m_hbm.at[stgi], sem).wait()
        _reset_stgi()

    def _append(kp, cur_id, a0, a1, a2, a3):
        rk = _splat_i32(kp)
        plsc.store_scatter(stg, (rk, iota), a0)
        plsc.store_scatter(stg, (rk, iota + 16), a1)
        plsc.store_scatter(stg, (rk, iota + 32), a2)
        plsc.store_scatter(stg, (rk, iota + 48), a3)
        plsc.store_scatter(stgi, (rk,), _splat_i32(cur_id))
        return kp + 1

    def _to_bnd(slot, cur_id, a0, a1, a2, a3):
        bndil[pl.ds(slot * 16, 16)] = _splat_i32(cur_id)
        bndrl[pl.ds(slot * H, 16)] = a0
        bndrl[pl.ds(slot * H + 16, 16)] = a1
        bndrl[pl.ds(slot * H + 32, 16)] = a2
        bndrl[pl.ds(slot * H + 48, 16)] = a3

    def tile_body(t, carry):
        base = row0 + t * TILE
        pltpu.sync_copy(h_hbm.at[pl.ds(base * H, TILE * H)], htile)
        pltpu.sync_copy(cl_hbm.at[pl.ds(base, TILE)], idst)

        ones = jnp.ones((16,), jnp.int32)

        def pres_body(gi, c):
            idv = idst[pl.ds(gi * 16, 16)]
            plsc.store_scatter(presl, (idv,), ones)
            return c
        lax.fori_loop(0, TILE // 16, pres_body, 0)

        def grp_body(gg, c2):
            goff = gg * 16
            idv = idst[pl.ds(goff, 16)]
            hbase = goff * H
            carry2 = c2
            for lane in range(16):
                cur_id, cnt, kpos, a0, a1, a2, a3 = carry2
                rid = idv[lane]
                off = hbase + lane * H
                v0 = htile[pl.ds(off, 16)]
                v1 = htile[pl.ds(off + 16, 16)]
                v2 = htile[pl.ds(off + 32, 16)]
                v3 = htile[pl.ds(off + 48, 16)]
                new = rid != cur_id

                def do_close(cnt, kpos, cur_id=cur_id, a0=a0, a1=a1, a2=a2,
                             a3=a3):
                    shared_first = (cnt == 0) & (cur_id == left_id)

                    def tb(kp):
                        _to_bnd(0, cur_id, a0, a1, a2, a3)
                        return kp

                    def tm(kp):
                        return _append(kp, cur_id, a0, a1, a2, a3)

                    kpos2 = lax.cond(shared_first, tb, tm, kpos)
                    return cnt + 1, kpos2

                cnt, kpos = lax.cond(new & (cur_id >= 0), do_close,
                                     lambda c, k: (c, k), cnt, kpos)

                def fl(k):
                    _flush()
                    return jnp.int32(0)

                kpos = lax.cond(kpos == KCAP, fl, lambda k: k, kpos)

                a0 = jnp.where(new, v0, jnp.maximum(a0, v0))
                a1 = jnp.where(new, v1, jnp.maximum(a1, v1))
                a2 = jnp.where(new, v2, jnp.maximum(a2, v2))
                a3 = jnp.where(new, v3, jnp.maximum(a3, v3))
                carry2 = (rid, cnt, kpos, a0, a1, a2, a3)
            return carry2

        return lax.fori_loop(0, TILE // 16, grp_body, carry)

    neg16 = jnp.full((16,), NEG, jnp.float32)
    init = (jnp.int32(-2), jnp.int32(0), jnp.int32(0),
            neg16, neg16, neg16, neg16)
    cur_id, cnt, kpos, a0, a1, a2, a3 = lax.fori_loop(0, NT, tile_body, init)

    last_shared = (cur_id == right_id) | (cur_id == left_id)

    def end_bnd(kp):
        _to_bnd(1, cur_id, a0, a1, a2, a3)
        return kp

    def end_m(kp):
        return _append(kp, cur_id, a0, a1, a2, a3)

    kpos = lax.cond(last_shared, end_bnd, end_m, kpos)

    @pl.when(kpos > 0)
    def _():
        _flush()

    pltpu.sync_copy(bndrl, bndr_hbm.at[pl.ds(wid * 2 * H, 2 * H)])
    pltpu.sync_copy(bndil, bndi_hbm.at[pl.ds(wid * 32, 32)])
    pltpu.sync_copy(presl.at[pl.ds(0, C)], pres_hbm.at[pl.ds(wid * C, C)])


@functools.partial(
    pl.kernel,
    out_type=jax.ShapeDtypeStruct((N, H), jnp.float32),
    mesh=_MESH,
    compiler_params=pltpu.CompilerParams(needs_layout_passes=False, use_tc_tiling_on_sc=False),
    scratch_types=[
        pltpu.VMEM((P,), jnp.int32),
        pltpu.VMEM((4, GCH, H), jnp.float32),
        pltpu.SemaphoreType.DMA,
        pltpu.SemaphoreType.DMA,
        pltpu.SemaphoreType.DMA,
        pltpu.SemaphoreType.DMA,
        pltpu.SemaphoreType.DMA,
        pltpu.SemaphoreType.DMA,
        pltpu.SemaphoreType.DMA,
        pltpu.SemaphoreType.DMA,
    ],
)
def _sc_gather(m_hbm, cl_hbm, out_hbm,
               idsl, ring, g0, g1, g2, g3, o0, o1, o2, o3):
    wid = lax.axis_index("s") * 2 + lax.axis_index("c")
    base = wid * P
    gsem = (g0, g1, g2, g3)
    osem = (o0, o1, o2, o3)

    pltpu.sync_copy(cl_hbm.at[pl.ds(base, P)], idsl)

    def round_body(r, c):
        cps = []
        for s in range(4):
            i = r * 4 + s

            @pl.when(r > 0)
            def _():
                pltpu.make_async_copy(
                    ring.at[s], out_hbm.at[pl.ds(base, GCH)], osem[s]).wait()

            idx = idsl.at[pl.ds(i * GCH, GCH)]
            cps.append(pltpu.async_copy(m_hbm.at[idx], ring.at[s], gsem[s]))
        for s in range(4):
            i = r * 4 + s
            cps[s].wait()
            pltpu.async_copy(
                ring.at[s], out_hbm.at[pl.ds(base + i * GCH, GCH)], osem[s])
        return c

    lax.fori_loop(0, NG // 4, round_body, 0)

    for s in range(4):
        pltpu.make_async_copy(
            ring.at[s], out_hbm.at[pl.ds(base, GCH)], osem[s]).wait()

    i = NG - 1
    idx = idsl.at[pl.ds(i * GCH, GCH)]
    pltpu.async_copy(m_hbm.at[idx], ring.at[0], g0).wait()
    pltpu.sync_copy(ring.at[0], out_hbm.at[pl.ds(base + i * GCH, GCH)])


def _seg_round(h, cl):
    mb, br, bi, pres = _sc_segmax(h.reshape(N * H), cl)
    bi2 = bi.reshape(2 * NW, 16)
    m = _patch(mb, br.reshape(2 * NW, H), bi2, bi2)
    return _sc_gather(m, cl)


def kernel(x, cluster, W1_0, b1_0, g_0, be_0, W2_0, b2_0,
           W1_1, b1_1, g_1, be_1, W2_1, b2_1,
           W1_2, b1_2, g_2, be_2, W2_2, b2_2, Wf, bf):
    r1 = lambda v: v.reshape(1, H)
    cl = cluster

    h1 = _mlp0(x, W1_0, r1(b1_0), r1(g_0), r1(be_0), W2_0, r1(b2_0))
    gv1 = _seg_round(h1, cl)

    h2 = _mlpcat(h1, gv1, W1_1[:H], W1_1[H:], r1(b1_1), r1(g_1), r1(be_1),
                 W2_1, r1(b2_1))
    gv2 = _seg_round(h2, cl)

    h3 = _mlpcat(h2, gv2, W1_2[:H], W1_2[H:], r1(b1_2), r1(g_2), r1(be_2),
                 W2_2, r1(b2_2))
    gv3 = _seg_round(h3, cl)

    f = _catlin(h3, gv3, Wf[:H], Wf[H:], r1(bf))
    mb, br, bi, pres = _sc_segmax(f.reshape(N * H), cl)
    bi2 = bi.reshape(2 * NW, 16)
    out = _patch_norm(mb, br.reshape(2 * NW, H), bi2, bi2,
                      pres.reshape(NW, C).T)
    return out


# speedup vs baseline: 1.3842x; 1.1874x over previous
import functools

import jax
import jax.numpy as jnp
from jax import lax
from jax.experimental import pallas as pl
from jax.experimental.pallas import tpu as pltpu
from jax.experimental.pallas import tpu_sc as plsc

N = 320000
C = 10000
H = 64
D_IN = 128

NW = 32
P = N // NW
TILE = 400
NT = P // TILE
KCAP = 64
GCH = 80
NG = P // GCH
NEG = -3.0e38

_MESH = plsc.VectorSubcoreMesh(core_axis_name="c", subcore_axis_name="s")


NP2 = N // 2


def _half_stats(t):
    r = t.shape[0]
    m1 = jnp.mean(t[:, :H], axis=1, keepdims=True)
    m2 = jnp.mean(t[:, H:], axis=1, keepdims=True)
    return jnp.concatenate([jnp.broadcast_to(m1, (r, H)),
                            jnp.broadcast_to(m2, (r, H))], axis=1)


def _ln_relu_mm(pre, g, be, W2, b2):
    mu = _half_stats(pre)
    d = pre - mu
    var = _half_stats(d * d)
    h = d * lax.rsqrt(var + 1e-5) * g + be
    h = jnp.maximum(h, 0.0)
    return jnp.dot(h, W2, preferred_element_type=jnp.float32) + b2


def _mlp0_kern(x_ref, W1, b1, g, be, W2, b2, out_ref):
    pre = jnp.dot(x_ref[...], W1[...], preferred_element_type=jnp.float32) + b1[...]
    out_ref[...] = _ln_relu_mm(pre, g[...], be[...], W2[...], b2[...])


def _mlpcat_kern(h_ref, gv_ref, W1a, W1b, b1, g, be, W2, b2, out_ref):
    pre = (jnp.dot(h_ref[...], W1a[...], preferred_element_type=jnp.float32)
           + jnp.dot(gv_ref[...], W1b[...], preferred_element_type=jnp.float32)
           + b1[...])
    out_ref[...] = _ln_relu_mm(pre, g[...], be[...], W2[...], b2[...])


def _catlin_kern(h_ref, gv_ref, Wa, Wb, b, out_ref):
    out_ref[...] = (jnp.dot(h_ref[...], Wa[...], preferred_element_type=jnp.float32)
                    + jnp.dot(gv_ref[...], Wb[...], preferred_element_type=jnp.float32)
                    + b[...])


RB = 640
GRID = NP2 // RB

_w = lambda r, c: pl.BlockSpec((r, c), lambda i: (0, 0))
_rows = lambda c: pl.BlockSpec((RB, c), lambda i: (i, 0))

_mlp0 = pl.pallas_call(
    _mlp0_kern,
    grid=(GRID,),
    in_specs=[_rows(2 * D_IN), _w(2 * D_IN, 2 * H), _w(1, 2 * H),
              _w(1, 2 * H), _w(1, 2 * H), _w(2 * H, 2 * H), _w(1, 2 * H)],
    out_specs=_rows(2 * H),
    out_shape=jax.ShapeDtypeStruct((NP2, 2 * H), jnp.float32),
)

_mlpcat = pl.pallas_call(
    _mlpcat_kern,
    grid=(GRID,),
    in_specs=[_rows(2 * H), _rows(2 * H), _w(2 * H, 2 * H), _w(2 * H, 2 * H),
              _w(1, 2 * H), _w(1, 2 * H), _w(1, 2 * H), _w(2 * H, 2 * H),
              _w(1, 2 * H)],
    out_specs=_rows(2 * H),
    out_shape=jax.ShapeDtypeStruct((NP2, 2 * H), jnp.float32),
)

_catlin = pl.pallas_call(
    _catlin_kern,
    grid=(GRID,),
    in_specs=[_rows(2 * H), _rows(2 * H), _w(2 * H, 2 * H), _w(2 * H, 2 * H),
              _w(1, 2 * H)],
    out_specs=_rows(2 * H),
    out_shape=jax.ShapeDtypeStruct((NP2, 2 * H), jnp.float32),
)


def _combine_bnd(bndr, bndi):
    rows = bndr
    ids = bndi
    for sh in (1, 2, 4, 8, 16, 32):
        prev_id = jnp.concatenate(
            [jnp.full((sh, ids.shape[1]), -9, jnp.int32), ids[:-sh]], axis=0)
        prev_r = jnp.concatenate(
            [jnp.full((sh, H), NEG, jnp.float32), rows[:-sh]], axis=0)
        same = (ids == prev_id)[:, :1]
        rows = jnp.where(same, jnp.maximum(rows, prev_r), rows)
    return rows


def _apply_patches(out_ref, rows, bndi_smem):
    for j in range(2 * NW):
        idj = bndi_smem[j, 0]
        nxt = bndi_smem[j + 1, 0] if j < 2 * NW - 1 else jnp.int32(-7)

        @pl.when((idj >= 0) & (idj != nxt))
        def _():
            out_ref[pl.ds(idj, 1), :] = rows[j:j + 1, :]


def _patch_kern(m_ref, bndr_ref, bndi_ref, bndi_smem, out_ref):
    out_ref[...] = m_ref[0:C, :]
    rows = _combine_bnd(bndr_ref[...], bndi_ref[...])
    _apply_patches(out_ref, rows, bndi_smem)


def _patch_norm_kern(m_ref, bndr_ref, bndi_ref, bndi_smem, pres_ref, out_ref):
    out_ref[...] = m_ref[0:C, :]
    rows = _combine_bnd(bndr_ref[...], bndi_ref[...])
    _apply_patches(out_ref, rows, bndi_smem)
    h = out_ref[...]
    nrm = jnp.sqrt(jnp.sum(h * h, axis=1, keepdims=True))
    ok = jnp.sum(pres_ref[...], axis=1, keepdims=True) > 0
    out_ref[...] = jnp.where(ok, h / jnp.maximum(nrm, 1e-12), 0.0)


def _full(shp):
    n = len(shp)
    return pl.BlockSpec(shp, lambda: (0,) * n)


_patch = pl.pallas_call(
    _patch_kern,
    in_specs=[_full((C + 1, H)), _full((2 * NW, H)), _full((2 * NW, 16)),
              pl.BlockSpec(memory_space=pltpu.SMEM)],
    out_specs=_full((C, H)),
    out_shape=jax.ShapeDtypeStruct((C, H), jnp.float32),
)

_patch_norm = pl.pallas_call(
    _patch_norm_kern,
    in_specs=[_full((C + 1, H)), _full((2 * NW, H)), _full((2 * NW, 16)),
              pl.BlockSpec(memory_space=pltpu.SMEM), _full((C, NW))],
    out_specs=_full((C, H)),
    out_shape=jax.ShapeDtypeStruct((C, H), jnp.float32),
)


def _splat_i32(v):
    return jnp.full((16,), v, jnp.int32)


@functools.partial(
    pl.kernel,
    out_type=(
        jax.ShapeDtypeStruct((C + 1, H), jnp.float32),
        jax.ShapeDtypeStruct((2 * NW * H,), jnp.float32),
        jax.ShapeDtypeStruct((2 * NW * 16,), jnp.int32),
        jax.ShapeDtypeStruct((NW * C,), jnp.int32),
    ),
    mesh=_MESH,
    compiler_params=pltpu.CompilerParams(needs_layout_passes=False, use_tc_tiling_on_sc=False),
    scratch_types=[
        pltpu.VMEM((TILE * H,), jnp.float32),
        pltpu.VMEM((TILE,), jnp.int32),
        pltpu.VMEM((KCAP, H), jnp.float32),
        pltpu.VMEM((KCAP,), jnp.int32),
        pltpu.VMEM((10112,), jnp.int32),
        pltpu.VMEM((2 * H,), jnp.float32),
        pltpu.VMEM((32,), jnp.int32),
        pltpu.VMEM((16,), jnp.int32),
        pltpu.SemaphoreType.DMA,
    ],
)
def _sc_segmax(h_hbm, cl_hbm, m_hbm, bndr_hbm, bndi_hbm, pres_hbm,
               htile, idst, stg, stgi, presl, bndrl, bndil, idbuf, sem):
    wid = lax.axis_index("s") * 2 + lax.axis_index("c")
    row0 = wid * P
    iota = lax.iota(jnp.int32, 16)

    def _reset_stgi():
        for j in range(KCAP // 16):
            stgi[pl.ds(j * 16, 16)] = _splat_i32(C)

    _reset_stgi()
    for j in range(2):
        bndil[pl.ds(j * 16, 16)] = _splat_i32(-1)
    for j in range(2 * H // 16):
        bndrl[pl.ds(j * 16, 16)] = jnp.full((16,), NEG, jnp.float32)

    def _zp(i, c):
        presl[pl.ds(i * 16, 16)] = jnp.zeros((16,), jnp.int32)
        return c
    lax.fori_loop(0, C // 16, _zp, 0)

    def _left():
        pltpu.sync_copy(cl_hbm.at[pl.ds(row0 - 16, 16)], idbuf)
        return idbuf[pl.ds(0, 16)][15]

    left_id = lax.cond(wid > 0, _left, lambda: jnp.int32(-1))

    def _right():
        pltpu.sync_copy(cl_hbm.at[pl.ds(row0 + P, 16)], idbuf)
        return idbuf[pl.ds(0, 16)][0]

    right_id = lax.cond(wid < NW - 1, _right, lambda: jnp.int32(-1))

    def _flush():
        pltpu.async_copy(stg, m_hbm.at[stgi], sem).wait()
        _reset_stgi()

    def _append(kp, cur_id, a0, a1, a2, a3):
        rk = _splat_i32(kp)
        plsc.store_scatter(stg, (rk, iota), a0)
        plsc.store_scatter(stg, (rk, iota + 16), a1)
        plsc.store_scatter(stg, (rk, iota + 32), a2)
        plsc.store_scatter(stg, (rk, iota + 48), a3)
        plsc.store_scatter(stgi, (rk,), _splat_i32(cur_id))
        return kp + 1

    def _to_bnd(slot, cur_id, a0, a1, a2, a3):
        bndil[pl.ds(slot * 16, 16)] = _splat_i32(cur_id)
        bndrl[pl.ds(slot * H, 16)] = a0
        bndrl[pl.ds(slot * H + 16, 16)] = a1
        bndrl[pl.ds(slot * H + 32, 16)] = a2
        bndrl[pl.ds(slot * H + 48, 16)] = a3

    def tile_body(t, carry):
        base = row0 + t * TILE
        pltpu.sync_copy(h_hbm.at[pl.ds(base * H, TILE * H)], htile)
        pltpu.sync_copy(cl_hbm.at[pl.ds(base, TILE)], idst)

        ones = jnp.ones((16,), jnp.int32)

        def pres_body(gi, c):
            idv = idst[pl.ds(gi * 16, 16)]
            plsc.store_scatter(presl, (idv,), ones)
            return c
        lax.fori_loop(0, TILE // 16, pres_body, 0)

        def grp_body(gg, c2):
            goff = gg * 16
            idv = idst[pl.ds(goff, 16)]
            hbase = goff * H
            carry2 = c2
            for lane in range(16):
                cur_id, cnt, kpos, a0, a1, a2, a3 = carry2
                rid = idv[lane]
                off = hbase + lane * H
                v0 = htile[pl.ds(off, 16)]
                v1 = htile[pl.ds(off + 16, 16)]
                v2 = htile[pl.ds(off + 32, 16)]
                v3 = htile[pl.ds(off + 48, 16)]
                new = rid != cur_id

                def do_close(cnt, kpos, cur_id=cur_id, a0=a0, a1=a1, a2=a2,
                             a3=a3):
                    shared_first = (cnt == 0) & (cur_id == left_id)

                    def tb(kp):
                        _to_bnd(0, cur_id, a0, a1, a2, a3)
                        return kp

                    def tm(kp):
                        return _append(kp, cur_id, a0, a1, a2, a3)

                    kpos2 = lax.cond(shared_first, tb, tm, kpos)
                    return cnt + 1, kpos2

                cnt, kpos = lax.cond(new & (cur_id >= 0), do_close,
                                     lambda c, k: (c, k), cnt, kpos)

                def fl(k):
                    _flush()
                    return jnp.int32(0)

                kpos = lax.cond(kpos == KCAP, fl, lambda k: k, kpos)

                a0 = jnp.where(new, v0, jnp.maximum(a0, v0))
                a1 = jnp.where(new, v1, jnp.maximum(a1, v1))
                a2 = jnp.where(new, v2, jnp.maximum(a2, v2))
                a3 = jnp.where(new, v3, jnp.maximum(a3, v3))
                carry2 = (rid, cnt, kpos, a0, a1, a2, a3)
            return carry2

        return lax.fori_loop(0, TILE // 16, grp_body, carry)

    neg16 = jnp.full((16,), NEG, jnp.float32)
    init = (jnp.int32(-2), jnp.int32(0), jnp.int32(0),
            neg16, neg16, neg16, neg16)
    cur_id, cnt, kpos, a0, a1, a2, a3 = lax.fori_loop(0, NT, tile_body, init)

    last_shared = (cur_id == right_id) | (cur_id == left_id)

    def end_bnd(kp):
        _to_bnd(1, cur_id, a0, a1, a2, a3)
        return kp

    def end_m(kp):
        return _append(kp, cur_id, a0, a1, a2, a3)

    kpos = lax.cond(last_shared, end_bnd, end_m, kpos)

    @pl.when(kpos > 0)
    def _():
        _flush()

    pltpu.sync_copy(bndrl, bndr_hbm.at[pl.ds(wid * 2 * H, 2 * H)])
    pltpu.sync_copy(bndil, bndi_hbm.at[pl.ds(wid * 32, 32)])
    pltpu.sync_copy(presl.at[pl.ds(0, C)], pres_hbm.at[pl.ds(wid * C, C)])


@functools.partial(
    pl.kernel,
    out_type=jax.ShapeDtypeStruct((N, H), jnp.float32),
    mesh=_MESH,
    compiler_params=pltpu.CompilerParams(needs_layout_passes=False, use_tc_tiling_on_sc=False),
    scratch_types=[
        pltpu.VMEM((P,), jnp.int32),
        pltpu.VMEM((4, GCH, H), jnp.float32),
        pltpu.SemaphoreType.DMA,
        pltpu.SemaphoreType.DMA,
        pltpu.SemaphoreType.DMA,
        pltpu.SemaphoreType.DMA,
        pltpu.SemaphoreType.DMA,
        pltpu.SemaphoreType.DMA,
        pltpu.SemaphoreType.DMA,
        pltpu.SemaphoreType.DMA,
    ],
)
def _sc_gather(m_hbm, cl_hbm, out_hbm,
               idsl, ring, g0, g1, g2, g3, o0, o1, o2, o3):
    wid = lax.axis_index("s") * 2 + lax.axis_index("c")
    base = wid * P
    gsem = (g0, g1, g2, g3)
    osem = (o0, o1, o2, o3)

    pltpu.sync_copy(cl_hbm.at[pl.ds(base, P)], idsl)

    def round_body(r, c):
        cps = []
        for s in range(4):
            i = r * 4 + s

            @pl.when(r > 0)
            def _():
                pltpu.make_async_copy(
                    ring.at[s], out_hbm.at[pl.ds(base, GCH)], osem[s]).wait()

            idx = idsl.at[pl.ds(i * GCH, GCH)]
            cps.append(pltpu.async_copy(m_hbm.at[idx], ring.at[s], gsem[s]))
        for s in range(4):
            i = r * 4 + s
            cps[s].wait()
            pltpu.async_copy(
                ring.at[s], out_hbm.at[pl.ds(base + i * GCH, GCH)], osem[s])
        return c

    lax.fori_loop(0, NG // 4, round_body, 0)

    for s in range(4):
        pltpu.make_async_copy(
            ring.at[s], out_hbm.at[pl.ds(base, GCH)], osem[s]).wait()

    i = NG - 1
    idx = idsl.at[pl.ds(i * GCH, GCH)]
    pltpu.async_copy(m_hbm.at[idx], ring.at[0], g0).wait()
    pltpu.sync_copy(ring.at[0], out_hbm.at[pl.ds(base + i * GCH, GCH)])


def _bd(A):
    z = jnp.zeros_like(A)
    return jnp.concatenate(
        [jnp.concatenate([A, z], axis=1), jnp.concatenate([z, A], axis=1)],
        axis=0)


def _t2(v):
    return jnp.tile(v.reshape(1, H), (1, 2))


def _seg_round(h_p, cl):
    mb, br, bi, pres = _sc_segmax(h_p.reshape(N * H), cl)
    bi2 = bi.reshape(2 * NW, 16)
    m = _patch(mb, br.reshape(2 * NW, H), bi2, bi2)
    return _sc_gather(m, cl).reshape(NP2, 2 * H)


def kernel(x, cluster, W1_0, b1_0, g_0, be_0, W2_0, b2_0,
           W1_1, b1_1, g_1, be_1, W2_1, b2_1,
           W1_2, b1_2, g_2, be_2, W2_2, b2_2, Wf, bf):
    cl = cluster
    x_p = x.reshape(NP2, 2 * D_IN)

    h1 = _mlp0(x_p, _bd(W1_0), _t2(b1_0), _t2(g_0), _t2(be_0), _bd(W2_0),
               _t2(b2_0))
    gv1 = _seg_round(h1, cl)

    h2 = _mlpcat(h1, gv1, _bd(W1_1[:H]), _bd(W1_1[H:]), _t2(b1_1), _t2(g_1),
                 _t2(be_1), _bd(W2_1), _t2(b2_1))
    gv2 = _seg_round(h2, cl)

    h3 = _mlpcat(h2, gv2, _bd(W1_2[:H]), _bd(W1_2[H:]), _t2(b1_2), _t2(g_2),
                 _t2(be_2), _bd(W2_2), _t2(b2_2))
    gv3 = _seg_round(h3, cl)

    f = _catlin(h3, gv3, _bd(Wf[:H]), _bd(Wf[H:]), _t2(bf))
    mb, br, bi, pres = _sc_segmax(f.reshape(N * H), cl)
    bi2 = bi.reshape(2 * NW, 16)
    out = _patch_norm(mb, br.reshape(2 * NW, H), bi2, bi2,
                      pres.reshape(NW, C).T)
    return out
